# baseline (speedup 1.0000x reference)
import functools

import jax
import jax.numpy as jnp
from jax import lax
from jax.experimental import pallas as pl
from jax.experimental.pallas import tpu as pltpu
from jax.experimental.pallas import tpu_sc as plsc

_N = 96
_SUB = 64
_D = (512 * 512) // _SUB
_NW = 32
_PER_W = (_N * _SUB) // _NW
_B = 8
_NBUF = 3
_NCHUNK = _PER_W // _B


def _sc_body(x_hbm, perm_hbm, out_hbm, perm_v, idx_v, bufs, insems, outsems):
    w = lax.axis_index("s") * 2 + lax.axis_index("c")
    base = w * _PER_W

    pltpu.sync_copy(perm_hbm, perm_v)

    shift = _SUB.bit_length() - 1
    for v in range(_PER_W // 16):
        g = lax.iota(jnp.int32, 16) + (base + v * 16)
        row = lax.shift_right_logical(g, shift)
        sub = lax.bitwise_and(g, _SUB - 1)
        pv = plsc.load_gather(perm_v, [row])
        idx_v[pl.ds(v * 16, 16)] = pv * _SUB + sub

    def start_in(c):
        s = c % _NBUF
        pltpu.async_copy(
            x_hbm.at[idx_v.at[pl.ds(c * _B, _B)]], bufs.at[s], insems.at[s]
        )

    def wait_in(c):
        s = c % _NBUF
        pltpu.make_async_copy(
            x_hbm.at[idx_v.at[pl.ds(c * _B, _B)]], bufs.at[s], insems.at[s]
        ).wait()

    def start_out(c):
        s = c % _NBUF
        pltpu.async_copy(
            bufs.at[s], out_hbm.at[pl.ds(base + c * _B, _B)], outsems.at[s]
        )

    def wait_out(c):
        s = c % _NBUF
        pltpu.make_async_copy(
            bufs.at[s], out_hbm.at[pl.ds(base + c * _B, _B)], outsems.at[s]
        ).wait()

    start_in(0)
    start_in(1)
    for c in range(_NCHUNK):
        wait_in(c)
        start_out(c)
        if c + 2 < _NCHUNK:
            if c >= 1:
                wait_out(c - 1)
            start_in(c + 2)
    for c in range(max(_NCHUNK - 3, 0), _NCHUNK):
        wait_out(c)


@functools.partial(jax.jit, static_argnames=())
def _sc_permute(x2, perm32):
    mesh = plsc.VectorSubcoreMesh(core_axis_name="c", subcore_axis_name="s")
    run = pl.kernel(
        _sc_body,
        out_type=jax.ShapeDtypeStruct((_N * _SUB, _D), jnp.float32),
        mesh=mesh,
        scratch_types=[
            pltpu.VMEM((_N,), jnp.int32),
            pltpu.VMEM((_PER_W,), jnp.int32),
            pltpu.VMEM((_NBUF, _B, _D), jnp.float32),
            pltpu.SemaphoreType.DMA((_NBUF,)),
            pltpu.SemaphoreType.DMA((_NBUF,)),
        ],
        compiler_params=pltpu.CompilerParams(needs_layout_passes=False),
    )
    return run(x2, perm32)


def kernel(x, perm):
    n, h, w = x.shape
    x2 = x.reshape(_N * _SUB, _D)
    out2 = _sc_permute(x2, perm.astype(jnp.int32))
    return out2.reshape(n, h, w)


# speedup vs baseline: 3.5388x; 3.5388x over previous
import functools

import jax
import jax.numpy as jnp
from jax import lax
from jax.experimental import pallas as pl
from jax.experimental.pallas import tpu as pltpu
from jax.experimental.pallas import tpu_sc as plsc

_N = 96
_SUB = 64
_BAND = 8
_W = 512
_NW = 32
_PER_W = (_N * _SUB) // _NW
_B = 8
_NBUF = 3
_NCHUNK = _PER_W // _B


def _sc_body(x_hbm, perm_hbm, out_hbm, perm_v, idx_v, bufs, insems, outsems):
    w = lax.axis_index("s") * 2 + lax.axis_index("c")
    base = w * _PER_W

    pltpu.sync_copy(perm_hbm, perm_v)

    shift = _SUB.bit_length() - 1
    for v in range(_PER_W // 16):
        g = lax.iota(jnp.int32, 16) + (base + v * 16)
        row = lax.shift_right_logical(g, shift)
        sub = lax.bitwise_and(g, _SUB - 1)
        pv = plsc.load_gather(perm_v, [row])
        idx_v[pl.ds(v * 16, 16)] = pv * _SUB + sub

    def start_in(c):
        s = c % _NBUF
        pltpu.async_copy(
            x_hbm.at[idx_v.at[pl.ds(c * _B, _B)]], bufs.at[s], insems.at[s]
        )

    def wait_in(c):
        s = c % _NBUF
        pltpu.make_async_copy(
            x_hbm.at[idx_v.at[pl.ds(c * _B, _B)]], bufs.at[s], insems.at[s]
        ).wait()

    def start_out(c):
        s = c % _NBUF
        pltpu.async_copy(
            bufs.at[s], out_hbm.at[pl.ds(base + c * _B, _B)], outsems.at[s]
        )

    def wait_out(c):
        s = c % _NBUF
        pltpu.make_async_copy(
            bufs.at[s], out_hbm.at[pl.ds(base + c * _B, _B)], outsems.at[s]
        ).wait()

    start_in(0)
    start_in(1)
    for c in range(_NCHUNK):
        wait_in(c)
        start_out(c)
        if c + 2 < _NCHUNK:
            if c >= 1:
                wait_out(c - 1)
            start_in(c + 2)
    for c in range(max(_NCHUNK - 3, 0), _NCHUNK):
        wait_out(c)


@functools.partial(jax.jit, static_argnames=())
def _sc_permute(x3, perm32):
    mesh = plsc.VectorSubcoreMesh(core_axis_name="c", subcore_axis_name="s")
    run = pl.kernel(
        _sc_body,
        out_type=jax.ShapeDtypeStruct((_N * _SUB, _BAND, _W), jnp.float32),
        mesh=mesh,
        scratch_types=[
            pltpu.VMEM((_N,), jnp.int32),
            pltpu.VMEM((_PER_W,), jnp.int32),
            pltpu.VMEM((_NBUF, _B, _BAND, _W), jnp.float32),
            pltpu.SemaphoreType.DMA((_NBUF,)),
            pltpu.SemaphoreType.DMA((_NBUF,)),
        ],
        compiler_params=pltpu.CompilerParams(needs_layout_passes=False),
    )
    return run(x3, perm32)


def kernel(x, perm):
    n, h, w = x.shape
    x3 = x.reshape(_N * _SUB, _BAND, _W)
    out3 = _sc_permute(x3, perm.astype(jnp.int32))
    return out3.reshape(n, h, w)
